# Initial kernel scaffold; baseline (speedup 1.0000x reference)
import jax
import jax.numpy as jnp
from jax.experimental import pallas as pl


def kernel(input_ids, token_type_ids, word_emb, pos_emb, type_emb, ln_gamma, ln_beta):
    raise NotImplementedError("write your pallas kernel here")


# speedup vs baseline: 2.0245x; 2.0245x over previous
import functools

import jax
import jax.numpy as jnp
from jax import lax
from jax.experimental import pallas as pl
from jax.experimental.pallas import tpu as pltpu
from jax.experimental.pallas import tpu_sc as plsc

D_MODEL = 768
LN_EPS = 1e-12


def _word_gather_sc(ids, word_emb, *, n_tokens):
    info = plsc.get_sparse_core_info()
    n_workers = info.num_cores * info.num_subcores
    tpw = n_tokens // n_workers
    chunk = 64
    n_chunks = tpw // chunk
    mesh = plsc.VectorSubcoreMesh(core_axis_name="c", subcore_axis_name="s")

    @functools.partial(
        pl.kernel,
        out_type=jax.ShapeDtypeStruct((n_tokens, D_MODEL), jnp.float32),
        mesh=mesh,
        scratch_types=[
            pltpu.VMEM((chunk,), jnp.int32),
            pltpu.VMEM((chunk, D_MODEL), jnp.float32),
            pltpu.SemaphoreType.DMA,
        ],
    )
    def gather_kernel(ids_hbm, word_hbm, out_hbm, idx_v, rows_v, sem):
        wid = lax.axis_index("s") * info.num_cores + lax.axis_index("c")
        base = wid * tpw
        for c in range(n_chunks):
            off = base + c * chunk
            pltpu.sync_copy(ids_hbm.at[pl.ds(off, chunk)], idx_v)
            pltpu.async_copy(word_hbm.at[idx_v], rows_v, sem).wait()
            pltpu.sync_copy(rows_v, out_hbm.at[pl.ds(off, chunk)])

    return gather_kernel(ids, word_emb)


def _add_ln_tc(x, tts3, pos_emb, type_emb, gamma, beta, *, seq_len):
    n = x.shape[0]
    blk = 256
    pos_blocks = seq_len // blk

    def body(x_ref, tt_ref, pos_ref, typ_ref, g_ref, b_ref, o_ref):
        ttf = tt_ref[0].astype(jnp.float32).reshape(blk, 1)
        t0 = typ_ref[0:1, :]
        t1 = typ_ref[1:2, :]
        xb = x_ref[...] + pos_ref[...] + t0 + ttf * (t1 - t0)
        mean = jnp.mean(xb, axis=-1, keepdims=True)
        xc = xb - mean
        var = jnp.mean(xc * xc, axis=-1, keepdims=True)
        o_ref[...] = xc * lax.rsqrt(var + LN_EPS) * g_ref[...] + b_ref[...]

    return pl.pallas_call(
        body,
        grid=(n // blk,),
        in_specs=[
            pl.BlockSpec((blk, D_MODEL), lambda i: (i, 0)),
            pl.BlockSpec((1, 1, blk), lambda i: (i, 0, 0)),
            pl.BlockSpec((blk, D_MODEL), lambda i: (i % pos_blocks, 0)),
            pl.BlockSpec((2, D_MODEL), lambda i: (0, 0)),
            pl.BlockSpec((1, D_MODEL), lambda i: (0, 0)),
            pl.BlockSpec((1, D_MODEL), lambda i: (0, 0)),
        ],
        out_specs=pl.BlockSpec((blk, D_MODEL), lambda i: (i, 0)),
        out_shape=jax.ShapeDtypeStruct((n, D_MODEL), jnp.float32),
    )(x, tts3, pos_emb, type_emb, gamma.reshape(1, D_MODEL), beta.reshape(1, D_MODEL))


def kernel(input_ids, token_type_ids, word_emb, pos_emb, type_emb, ln_gamma, ln_beta):
    b, s = input_ids.shape
    n = b * s
    blk = 256
    ids = input_ids.reshape(n).astype(jnp.int32)
    tts3 = token_type_ids.reshape(n // blk, 1, blk).astype(jnp.int32)
    gathered = _word_gather_sc(ids, word_emb, n_tokens=n)
    out = _add_ln_tc(gathered, tts3, pos_emb, type_emb, ln_gamma, ln_beta,
                     seq_len=s)
    return out.reshape(b, s, D_MODEL)


# speedup vs baseline: 2.3690x; 1.1701x over previous
import functools

import jax
import jax.numpy as jnp
from jax import lax
from jax.experimental import pallas as pl
from jax.experimental.pallas import tpu as pltpu
from jax.experimental.pallas import tpu_sc as plsc

D_MODEL = 768
LN_EPS = 1e-12


def _word_gather_sc(ids, word_emb, *, n_tokens):
    info = plsc.get_sparse_core_info()
    n_workers = info.num_cores * info.num_subcores
    tpw = n_tokens // n_workers
    chunk = 64
    n_chunks = tpw // chunk
    mesh = plsc.VectorSubcoreMesh(core_axis_name="c", subcore_axis_name="s")

    @functools.partial(
        pl.kernel,
        out_type=jax.ShapeDtypeStruct((n_tokens, D_MODEL), jnp.float32),
        mesh=mesh,
        scratch_types=[
            pltpu.VMEM((tpw,), jnp.int32),
            pltpu.VMEM((2, chunk, D_MODEL), jnp.float32),
            pltpu.SemaphoreType.DMA,
            pltpu.SemaphoreType.DMA,
            pltpu.SemaphoreType.DMA,
            pltpu.SemaphoreType.DMA,
        ],
    )
    def gather_kernel(ids_hbm, word_hbm, out_hbm, idx_v, rows_v,
                      sem_g0, sem_g1, sem_w0, sem_w1):
        wid = lax.axis_index("s") * info.num_cores + lax.axis_index("c")
        base = wid * tpw
        sems_g = (sem_g0, sem_g1)
        sems_w = (sem_w0, sem_w1)
        pltpu.sync_copy(ids_hbm.at[pl.ds(base, tpw)], idx_v)

        def start_gather(c, buf):
            return pltpu.async_copy(
                word_hbm.at[idx_v.at[pl.ds(c * chunk, chunk)]],
                rows_v.at[buf], sems_g[buf])

        def start_write(c, buf):
            return pltpu.async_copy(
                rows_v.at[buf], out_hbm.at[pl.ds(base + c * chunk, chunk)],
                sems_w[buf])

        gathers = [start_gather(0, 0), None]
        writes = [None, None]
        for c in range(n_chunks):
            buf = c % 2
            gathers[buf].wait()
            writes[buf] = start_write(c, buf)
            nc = c + 1
            if nc < n_chunks:
                nbuf = nc % 2
                if writes[nbuf] is not None:
                    writes[nbuf].wait()
                gathers[nbuf] = start_gather(nc, nbuf)
        for buf in (0, 1):
            if writes[buf] is not None:
                writes[buf].wait()

    return gather_kernel(ids, word_emb)


def _add_ln_tc(x, tts3, pos_emb, type_emb, gamma, beta, *, seq_len, blk):
    n = x.shape[0]
    pos_blocks = seq_len // blk
    batch = n // seq_len

    def body(x_ref, tt_ref, pos_ref, typ_ref, g_ref, b_ref, o_ref):
        ttf = tt_ref[0].astype(jnp.float32).reshape(blk, 1)
        t0 = typ_ref[0:1, :]
        t1 = typ_ref[1:2, :]
        xb = x_ref[...] + pos_ref[...] + t0 + ttf * (t1 - t0)
        mean = jnp.mean(xb, axis=-1, keepdims=True)
        xc = xb - mean
        var = jnp.mean(xc * xc, axis=-1, keepdims=True)
        o_ref[...] = xc * lax.rsqrt(var + LN_EPS) * g_ref[...] + b_ref[...]

    return pl.pallas_call(
        body,
        grid=(pos_blocks, batch),
        in_specs=[
            pl.BlockSpec((blk, D_MODEL), lambda j, i: (i * pos_blocks + j, 0)),
            pl.BlockSpec((1, 1, blk), lambda j, i: (i * pos_blocks + j, 0, 0)),
            pl.BlockSpec((blk, D_MODEL), lambda j, i: (j, 0)),
            pl.BlockSpec((2, D_MODEL), lambda j, i: (0, 0)),
            pl.BlockSpec((1, D_MODEL), lambda j, i: (0, 0)),
            pl.BlockSpec((1, D_MODEL), lambda j, i: (0, 0)),
        ],
        out_specs=pl.BlockSpec((blk, D_MODEL), lambda j, i: (i * pos_blocks + j, 0)),
        out_shape=jax.ShapeDtypeStruct((n, D_MODEL), jnp.float32),
    )(x, tts3, pos_emb, type_emb, gamma.reshape(1, D_MODEL), beta.reshape(1, D_MODEL))


def kernel(input_ids, token_type_ids, word_emb, pos_emb, type_emb, ln_gamma, ln_beta):
    b, s = input_ids.shape
    n = b * s
    blk = 512
    ids = input_ids.reshape(n).astype(jnp.int32)
    tts3 = token_type_ids.reshape(n // blk, 1, blk).astype(jnp.int32)
    gathered = _word_gather_sc(ids, word_emb, n_tokens=n)
    out = _add_ln_tc(gathered, tts3, pos_emb, type_emb, ln_gamma, ln_beta,
                     seq_len=s, blk=blk)
    return out.reshape(b, s, D_MODEL)
